# baseline (speedup 1.0000x reference)
import functools
import jax
import jax.numpy as jnp
from jax import lax
from jax.experimental import pallas as pl
from jax.experimental.pallas import tpu as pltpu
from jax.experimental.pallas import tpu_sc as plsc

R = 5
DIN = 128
D = 64
C = 128
EPAD = 200064
RNG = 12512
ACCR = 12800
NT = 16


def _transform_body(x_ref, wrev_ref, brev_ref, wagg_ref, out_ref):
    x = x_ref[...]
    blk = x.shape[0]
    for r in range(R):
        m = jnp.dot(x, wrev_ref[r], preferred_element_type=jnp.float32) + brev_ref[r]
        p = jnp.dot(m, wagg_ref[pl.ds(D * (r + 1), D), :],
                    preferred_element_type=jnp.float32)
        out_ref[r] = jnp.concatenate(
            [p, jnp.ones((blk, 1), jnp.float32),
             jnp.zeros((blk, DIN - D - 1), jnp.float32)], axis=1)


def _item_transform(x, wrev, brev, wagg, block=2000):
    n = x.shape[0]
    return pl.pallas_call(
        _transform_body,
        grid=(n // block,),
        in_specs=[
            pl.BlockSpec((block, DIN), lambda i: (i, 0)),
            pl.BlockSpec((R, DIN, D), lambda i: (0, 0, 0)),
            pl.BlockSpec((R, D), lambda i: (0, 0)),
            pl.BlockSpec((D * (R + 1), D), lambda i: (0, 0)),
        ],
        out_specs=pl.BlockSpec((R, block, DIN), lambda i: (0, i, 0)),
        out_shape=jax.ShapeDtypeStruct((R, n, DIN), jnp.float32),
    )(x, wrev, brev, wagg)


def _final_body(uf_ref, s_ref, wagg_ref, bagg_ref, out_ref):
    acc = jnp.dot(uf_ref[...], wagg_ref[pl.ds(0, D), :],
                  preferred_element_type=jnp.float32)
    lanes = lax.broadcasted_iota(jnp.int32, (1, DIN), 1)
    cnt_hot = jnp.where(lanes == D, 1.0, 0.0)
    for r in range(R):
        srow = s_ref[r]
        cnt = jnp.sum(srow * cnt_hot, axis=1, keepdims=True)
        inv = 1.0 / jnp.maximum(cnt, 1.0)
        acc = acc + srow[:, :D] * inv
    acc = acc + bagg_ref[...]
    out_ref[...] = jnp.where(acc >= 0, acc, 0.01 * acc)


def _final(uf, s, wagg, bagg, block=2000):
    u = uf.shape[0]
    return pl.pallas_call(
        _final_body,
        grid=(u // block,),
        in_specs=[
            pl.BlockSpec((block, D), lambda i: (i, 0)),
            pl.BlockSpec((R, block, DIN), lambda i: (0, i, 0)),
            pl.BlockSpec((D * (R + 1), D), lambda i: (0, 0)),
            pl.BlockSpec((1, D), lambda i: (0, 0)),
        ],
        out_specs=pl.BlockSpec((block, D), lambda i: (i, 0)),
        out_shape=jax.ShapeDtypeStruct((u, D), jnp.float32),
    )(uf, s, wagg, bagg)


SOUT = 8 * RNG


def _edge_stage(p, src1, dst1, n_users):
    nitems = p.shape[1]
    p = p.reshape(R * nitems, DIN)
    nchunks = EPAD // C
    iters_long = nchunks - NT * (nchunks // NT)
    mesh = plsc.VectorSubcoreMesh(core_axis_name="c", subcore_axis_name="s")

    @functools.partial(
        pl.kernel, mesh=mesh,
        out_type=jax.ShapeDtypeStruct((R, SOUT, DIN), jnp.float32),
        scratch_types=[
            pltpu.VMEM_SHARED((ACCR, DIN), jnp.float32),
            pltpu.VMEM((C,), jnp.int32),
            pltpu.VMEM((C,), jnp.int32),
            pltpu.VMEM((C, DIN), jnp.float32),
            pltpu.SemaphoreType.DMA,
        ],
    )
    def k_sc(p_hbm, src_hbm, dst_hbm, s_hbm, acc, src_v, dst_v, rows_v, sem):
        c = lax.axis_index("c")
        s = lax.axis_index("s")
        zvec = jnp.zeros((16,), jnp.float32)

        nhi = jnp.where(s < iters_long, nchunks // NT + 1, nchunks // NT)
        row0 = s * 800

        def zero_acc():
            def zfill(i, carry):
                for k in range(DIN // 16):
                    rows_v[i, pl.ds(k * 16, 16)] = zvec
                return carry
            lax.fori_loop(0, C, zfill, 0)
            for k in range(6):
                pltpu.sync_copy(rows_v, acc.at[pl.ds(row0 + C * k, C)])
            pltpu.sync_copy(rows_v.at[pl.ds(0, 32)],
                            acc.at[pl.ds(row0 + 6 * C, 32)])

        def remap(base, rbase):
            def body(k, carry):
                d = dst_v[pl.ds(k * 16, 16)]
                loc = d - base
                ok = (loc >= 0) & (loc < RNG)
                dst_v[pl.ds(k * 16, 16)] = jnp.where(ok, loc, RNG)
                src_v[pl.ds(k * 16, 16)] = src_v[pl.ds(k * 16, 16)] + rbase
                return carry
            lax.fori_loop(0, C // 16, body, 0)

        def edge_chunks(r, base, rbase):
            def body(i, carry):
                j = s + NT * i
                off = pl.multiple_of(r * EPAD + j * C, C)
                pltpu.sync_copy(dst_hbm.at[pl.ds(off, C)], dst_v)
                pltpu.sync_copy(src_hbm.at[pl.ds(off, C)], src_v)
                remap(base, rbase)
                pltpu.async_copy(p_hbm.at[src_v], rows_v, sem).wait()
                pltpu.sync_copy(rows_v, acc.at[dst_v], add=True)
                return carry
            lax.fori_loop(0, nhi, body, 0)

        def drain(r, u0):
            @pl.when(s < 15)
            def _():
                for k in range(4):
                    off = pl.multiple_of(row0 + 200 * k, 8)
                    pltpu.sync_copy(acc.at[pl.ds(off, 200)],
                                    s_hbm.at[r, pl.ds(u0 + off, 200)])

            @pl.when(s == 15)
            def _():
                for off0, ln in ((0, 200), (200, 200), (400, 112)):
                    off = pl.multiple_of(12000 + off0, 8)
                    pltpu.sync_copy(acc.at[pl.ds(off, ln)],
                                    s_hbm.at[r, pl.ds(u0 + off, ln)])

        for gg in range(4):
            base = (c * 4 + gg) * RNG
            for r in range(R):
                zero_acc()
                plsc.subcore_barrier()
                edge_chunks(r, base, r * nitems)
                plsc.subcore_barrier()
                drain(r, base)

    return k_sc(p, src1, dst1)


def kernel(item_features, user_features, item_nids,
           edge_src_0, edge_dst_0, edge_src_1, edge_dst_1,
           edge_src_2, edge_dst_2, edge_src_3, edge_dst_3,
           edge_src_4, edge_dst_4,
           item_id_table,
           W_rev_0, b_rev_0, W_rev_1, b_rev_1, W_rev_2, b_rev_2,
           W_rev_3, b_rev_3, W_rev_4, b_rev_4,
           W_agg, b_agg):
    n_users = user_features.shape[0]
    e = edge_src_0.shape[0]
    item_id_emb = jnp.take(item_id_table, item_nids, axis=0)
    x = jnp.concatenate([item_features, item_id_emb], axis=1)
    wrev = jnp.stack([W_rev_0, W_rev_1, W_rev_2, W_rev_3, W_rev_4])
    brev = jnp.stack([b_rev_0, b_rev_1, b_rev_2, b_rev_3, b_rev_4])

    p = _item_transform(x, wrev, brev, W_agg)

    spad = jnp.zeros((EPAD - e,), jnp.int32)
    dpad = jnp.full((EPAD - e,), 1 << 20, jnp.int32)
    src1 = jnp.concatenate([
        edge_src_0.astype(jnp.int32), spad, edge_src_1.astype(jnp.int32), spad,
        edge_src_2.astype(jnp.int32), spad, edge_src_3.astype(jnp.int32), spad,
        edge_src_4.astype(jnp.int32), spad])
    dst1 = jnp.concatenate([
        edge_dst_0.astype(jnp.int32), dpad, edge_dst_1.astype(jnp.int32), dpad,
        edge_dst_2.astype(jnp.int32), dpad, edge_dst_3.astype(jnp.int32), dpad,
        edge_dst_4.astype(jnp.int32), dpad])

    s = _edge_stage(p, src1, dst1, n_users)

    return _final(user_features, s, W_agg, b_agg.reshape(1, D))
